# Initial kernel scaffold; baseline (speedup 1.0000x reference)
import jax
import jax.numpy as jnp
from jax.experimental import pallas as pl


def kernel(h, r, pos_t, neg_t, user_entity_embed, relation_embed, W_R):
    raise NotImplementedError("write your pallas kernel here")


# speedup vs baseline: 1.3367x; 1.3367x over previous
import jax
import jax.numpy as jnp
from jax import lax
from jax.experimental import pallas as pl
from jax.experimental.pallas import tpu as pltpu
from jax.experimental.pallas import tpu_sc as plsc

_B = 16384
_D = 64
_R = 64
_NC = 2
_NS = 16
_NW = _NC * _NS
_IDXW = 128

_N_IDX = 3 * _B
_CHUNKS_PER_W = _N_IDX // (_NW * _IDXW)


_ROWS_PER_W = _CHUNKS_PER_W * _IDXW


def _sc_gather_body(table_ref, idx_ref, out_ref, idx_v, rows_v, sem):
    wid = lax.axis_index("s") * _NC + lax.axis_index("c")
    base = wid * _ROWS_PER_W
    pltpu.sync_copy(idx_ref.at[pl.ds(base, _ROWS_PER_W)], idx_v)
    descs = []
    for c in range(_CHUNKS_PER_W):
        descs.append(
            pltpu.async_copy(
                table_ref.at[idx_v.at[pl.ds(c * _IDXW, _IDXW)]],
                rows_v.at[pl.ds(c * _IDXW, _IDXW)],
                sem,
            )
        )
    for d in descs:
        d.wait()
    pltpu.sync_copy(rows_v, out_ref.at[pl.ds(base, _ROWS_PER_W)])


def _sc_gather(table, idx1d):
    mesh = plsc.VectorSubcoreMesh(core_axis_name="c", subcore_axis_name="s")
    f = pl.kernel(
        _sc_gather_body,
        out_type=jax.ShapeDtypeStruct((_N_IDX, _D), jnp.float32),
        mesh=mesh,
        scratch_types=[
            pltpu.VMEM((_ROWS_PER_W,), jnp.int32),
            pltpu.VMEM((_ROWS_PER_W, _D), jnp.float32),
            pltpu.SemaphoreType.DMA,
        ],
        compiler_params=pltpu.CompilerParams(use_tc_tiling_on_sc=False),
    )
    return f(table, idx1d)


_BB = 512
_GRID = _B // _BB


def _tc_loss_body(r_ref, eh_ref, ep_ref, en_ref, wf_ref, re_ref, out_ref):
    i = pl.program_id(0)
    rb = r_ref[...]
    onehot = rb == lax.broadcasted_iota(jnp.int32, (_BB, _R), 1)
    o_f = onehot.astype(jnp.float32)
    o_b = onehot.astype(jnp.bfloat16)

    re = jnp.dot(o_f, re_ref[...], preferred_element_type=jnp.float32)

    wf = wf_ref[...]

    def proj(e_ref):
        e = e_ref[...].astype(jnp.bfloat16)
        h2 = (o_b[:, :, None] * e[:, None, :]).reshape(_BB, _R * _D)
        return jnp.dot(h2, wf, preferred_element_type=jnp.float32)

    rm_h = proj(eh_ref)
    rm_p = proj(ep_ref)
    rm_n = proj(en_ref)

    pos = jnp.sum(jnp.square(rm_h + re - rm_p), axis=1, keepdims=True)
    neg = jnp.sum(jnp.square(rm_h + re - rm_n), axis=1, keepdims=True)
    z = neg - pos
    ls = jnp.minimum(z, 0.0) - jnp.log(1.0 + jnp.exp(-jnp.abs(z)))

    s_ls = jnp.sum(ls)
    l2h = jnp.sum(rm_h * rm_h)
    l2re = jnp.sum(re * re)
    l2p = jnp.sum(rm_p * rm_p)
    l2n = jnp.sum(rm_n * rm_n)

    li = lax.broadcasted_iota(jnp.int32, (8, 128), 1)
    si = lax.broadcasted_iota(jnp.int32, (8, 128), 0)
    row0 = si == 0
    vec = (
        jnp.where(row0 & (li == 0), s_ls, 0.0)
        + jnp.where(row0 & (li == 1), l2h, 0.0)
        + jnp.where(row0 & (li == 2), l2re, 0.0)
        + jnp.where(row0 & (li == 3), l2p, 0.0)
        + jnp.where(row0 & (li == 4), l2n, 0.0)
    )

    @pl.when(i == 0)
    def _():
        out_ref[...] = jnp.zeros_like(out_ref)

    out_ref[...] += vec


def _tc_loss(r2, eh, ep, en, wflat_b, rel_embed):
    return pl.pallas_call(
        _tc_loss_body,
        grid=(_GRID,),
        in_specs=[
            pl.BlockSpec((_BB, 1), lambda i: (i, 0)),
            pl.BlockSpec((_BB, _D), lambda i: (i, 0)),
            pl.BlockSpec((_BB, _D), lambda i: (i, 0)),
            pl.BlockSpec((_BB, _D), lambda i: (i, 0)),
            pl.BlockSpec((_R * _D, _D), lambda i: (0, 0)),
            pl.BlockSpec((_R, _D), lambda i: (0, 0)),
        ],
        out_specs=pl.BlockSpec((8, 128), lambda i: (0, 0)),
        out_shape=jax.ShapeDtypeStruct((8, 128), jnp.float32),
    )(r2, eh, ep, en, wflat_b, rel_embed)


def kernel(h, r, pos_t, neg_t, user_entity_embed, relation_embed, W_R):
    idx = jnp.concatenate([h, pos_t, neg_t]).astype(jnp.int32)
    rows = _sc_gather(user_entity_embed, idx)
    eh = rows[:_B]
    ep = rows[_B:2 * _B]
    en = rows[2 * _B:]

    wflat_b = W_R.reshape(_R * _D, _D).astype(jnp.bfloat16)
    r2 = r.astype(jnp.int32).reshape(_B, 1)
    out = _tc_loss(r2, eh, ep, en, wflat_b, relation_embed)

    o = out[0]
    kg_loss = -(o[0] / _B)
    l2_loss = (o[1] + o[2] + o[3] + o[4]) / (2.0 * _B)
    return kg_loss + 1e-5 * l2_loss


# speedup vs baseline: 1.7149x; 1.2829x over previous
import jax
import jax.numpy as jnp
from jax import lax
from jax.experimental import pallas as pl
from jax.experimental.pallas import tpu as pltpu
from jax.experimental.pallas import tpu_sc as plsc

_B = 16384
_D = 64
_R = 64
_G = 128
_P = 24576
_NBLK = _P // _G

_PB = 512
_NPB = _B // _PB

_NC = 2
_NS = 16
_NW = _NC * _NS
_IDXW = 128
_N_IDX = 3 * _B
_CHUNKS_PER_W = _N_IDX // (_NW * _IDXW)
_ROWS_PER_W = _CHUNKS_PER_W * _IDXW


def _pos_body(r_ref, l_ref, dh_ref, dp_ref, dn_ref, brel_ref, bnv_ref,
              cnt_s, pos_s):
    i = pl.program_id(0)

    @pl.when(i == 0)
    def _():
        cnt_s[...] = jnp.zeros_like(cnt_s)

    @pl.when(i < _NPB)
    def _pass_a():
        rb = r_ref[...]
        oh = rb == lax.broadcasted_iota(jnp.int32, (_PB, _R), 1)
        o_b = oh.astype(jnp.bfloat16)
        o_f = oh.astype(jnp.float32)
        c = jnp.dot(l_ref[...], o_b, preferred_element_type=jnp.float32)
        carry = cnt_s[...]
        pos = jnp.sum(o_f * (carry + c), axis=1, keepdims=True) - 1.0
        off = pl.multiple_of(i * _PB, _PB)
        pos_s[pl.ds(off, _PB), :] = pos
        cnt_s[...] = carry + c[_PB - 1:_PB, :]

    @pl.when(i >= _NPB)
    def _pass_b():
        j = i - _NPB
        cnt = cnt_s[...]
        padc = jnp.floor((cnt + (_G - 1.0)) / _G) * _G
        tri = (lax.broadcasted_iota(jnp.int32, (_R, _R), 0)
               < lax.broadcasted_iota(jnp.int32, (_R, _R), 1)).astype(jnp.float32)
        starts = jnp.dot(padc, tri, preferred_element_type=jnp.float32)

        rb = r_ref[...]
        oh = rb == lax.broadcasted_iota(jnp.int32, (_PB, _R), 1)
        o_f = oh.astype(jnp.float32)
        segstart = jnp.sum(o_f * starts, axis=1, keepdims=True)
        off = pl.multiple_of(j * _PB, _PB)
        d = segstart + pos_s[pl.ds(off, _PB), :]
        dh_ref[...] = d.astype(jnp.int32)
        dp_ref[...] = (d + float(_P)).astype(jnp.int32)
        dn_ref[...] = (d + float(2 * _P)).astype(jnp.int32)

        @pl.when(j == _NPB - 1)
        def _blocks():
            jr = (lax.broadcasted_iota(jnp.int32, (_NBLK, _R), 0) * _G
                  ).astype(jnp.float32)
            relc = lax.broadcasted_iota(jnp.int32, (_NBLK, _R), 1
                                        ).astype(jnp.float32)
            m = (starts <= jr) & (jr < starts + padc)
            m_f = m.astype(jnp.float32)
            brel_ref[...] = jnp.sum(m_f * relc, axis=1, keepdims=True
                                    ).astype(jnp.int32)
            nv = jnp.clip(cnt - (jr - starts), 0.0, float(_G))
            bnv_ref[...] = jnp.sum(m_f * nv, axis=1, keepdims=True
                                   ).astype(jnp.int32)


def _positions(r2, ltri):
    return pl.pallas_call(
        _pos_body,
        grid=(2 * _NPB,),
        in_specs=[
            pl.BlockSpec((_PB, 1), lambda i: (i % _NPB, 0)),
            pl.BlockSpec((_PB, _PB), lambda i: (0, 0)),
        ],
        out_specs=[
            pl.BlockSpec((_PB, 1), lambda i: (i % _NPB, 0)),
            pl.BlockSpec((_PB, 1), lambda i: (i % _NPB, 0)),
            pl.BlockSpec((_PB, 1), lambda i: (i % _NPB, 0)),
            pl.BlockSpec((_NBLK, 1), lambda i: (0, 0)),
            pl.BlockSpec((_NBLK, 1), lambda i: (0, 0)),
        ],
        out_shape=[
            jax.ShapeDtypeStruct((_B, 1), jnp.int32),
            jax.ShapeDtypeStruct((_B, 1), jnp.int32),
            jax.ShapeDtypeStruct((_B, 1), jnp.int32),
            jax.ShapeDtypeStruct((_NBLK, 1), jnp.int32),
            jax.ShapeDtypeStruct((_NBLK, 1), jnp.int32),
        ],
        scratch_shapes=[
            pltpu.VMEM((1, _R), jnp.float32),
            pltpu.VMEM((_B, 1), jnp.float32),
        ],
    )(r2, ltri)


def _sc_gs_body(table_ref, idx_ref, dst_ref, out_ref, idx_v, dst_v, rows_v,
                gsem, ssem):
    wid = lax.axis_index("s") * _NC + lax.axis_index("c")
    pltpu.sync_copy(idx_ref.at[wid], idx_v)
    pltpu.sync_copy(dst_ref.at[wid], dst_v)
    gd = []
    for c in range(_CHUNKS_PER_W):
        gd.append(
            pltpu.async_copy(
                table_ref.at[idx_v.at[c]],
                rows_v.at[pl.ds(c * _IDXW, _IDXW)],
                gsem,
            )
        )
    for d in gd:
        d.wait()
    sd = []
    for c in range(_CHUNKS_PER_W):
        sd.append(
            pltpu.async_copy(
                rows_v.at[pl.ds(c * _IDXW, _IDXW)],
                out_ref.at[dst_v.at[c]],
                ssem,
            )
        )
    for d in sd:
        d.wait()


def _sc_gather_scatter(table, idx3, dst3):
    mesh = plsc.VectorSubcoreMesh(core_axis_name="c", subcore_axis_name="s")
    f = pl.kernel(
        _sc_gs_body,
        out_type=jax.ShapeDtypeStruct((3 * _P, _D), jnp.float32),
        mesh=mesh,
        scratch_types=[
            pltpu.VMEM((_CHUNKS_PER_W, _IDXW), jnp.int32),
            pltpu.VMEM((_CHUNKS_PER_W, _IDXW), jnp.int32),
            pltpu.VMEM((_ROWS_PER_W, _D), jnp.float32),
            pltpu.SemaphoreType.DMA,
            pltpu.SemaphoreType.DMA,
        ],
        compiler_params=pltpu.CompilerParams(use_tc_tiling_on_sc=False),
    )
    return f(table, idx3, dst3)


def _loss_body(brel_ref, bnv_ref, eh_ref, ep_ref, en_ref, w_ref, re_ref,
               out_ref):
    i = pl.program_id(0)
    nv = bnv_ref[i]

    w = w_ref[0].astype(jnp.bfloat16)
    e3 = jnp.concatenate([eh_ref[...], ep_ref[...], en_ref[...]], axis=0)
    rm3 = jnp.dot(e3.astype(jnp.bfloat16), w, preferred_element_type=jnp.float32)
    rm_h = rm3[:_G]
    rm_p = rm3[_G:2 * _G]
    rm_n = rm3[2 * _G:]
    re_row = re_ref[0]

    mrow = lax.broadcasted_iota(jnp.int32, (_G, 1), 0) < nv

    a = rm_h + re_row
    pos = jnp.sum(jnp.square(a - rm_p), axis=1, keepdims=True)
    neg = jnp.sum(jnp.square(a - rm_n), axis=1, keepdims=True)
    z = neg - pos
    ls = jnp.minimum(z, 0.0) - jnp.log(1.0 + jnp.exp(-jnp.abs(z)))

    s_ls = jnp.sum(jnp.where(mrow, ls, jnp.zeros_like(ls)))
    m2 = jnp.broadcast_to(mrow, (_G, _D))
    zz = jnp.zeros((_G, _D), jnp.float32)
    l2h = jnp.sum(jnp.where(m2, rm_h * rm_h, zz))
    l2p = jnp.sum(jnp.where(m2, rm_p * rm_p, zz))
    l2n = jnp.sum(jnp.where(m2, rm_n * rm_n, zz))
    l2re = nv.astype(jnp.float32) * jnp.sum(re_row * re_row)

    li = lax.broadcasted_iota(jnp.int32, (8, 128), 1)
    si = lax.broadcasted_iota(jnp.int32, (8, 128), 0)
    row0 = si == 0
    vec = (
        jnp.where(row0 & (li == 0), s_ls, 0.0)
        + jnp.where(row0 & (li == 1), l2h, 0.0)
        + jnp.where(row0 & (li == 2), l2re, 0.0)
        + jnp.where(row0 & (li == 3), l2p, 0.0)
        + jnp.where(row0 & (li == 4), l2n, 0.0)
    )

    @pl.when(i == 0)
    def _():
        out_ref[...] = jnp.zeros_like(out_ref)

    out_ref[...] += vec


def _grouped_loss(brel, bnv, eh_s, ep_s, en_s, W_R, rel_embed):
    grid_spec = pltpu.PrefetchScalarGridSpec(
        num_scalar_prefetch=2,
        grid=(_NBLK,),
        in_specs=[
            pl.BlockSpec((_G, _D), lambda i, brel, bnv: (i, 0)),
            pl.BlockSpec((_G, _D), lambda i, brel, bnv: (i, 0)),
            pl.BlockSpec((_G, _D), lambda i, brel, bnv: (i, 0)),
            pl.BlockSpec((1, _D, _D), lambda i, brel, bnv: (brel[i], 0, 0)),
            pl.BlockSpec((1, 1, _D), lambda i, brel, bnv: (brel[i], 0, 0)),
        ],
        out_specs=pl.BlockSpec((8, 128), lambda i, brel, bnv: (0, 0)),
    )
    return pl.pallas_call(
        _loss_body,
        grid_spec=grid_spec,
        out_shape=jax.ShapeDtypeStruct((8, 128), jnp.float32),
    )(brel, bnv, eh_s, ep_s, en_s, W_R, rel_embed.reshape(_R, 1, _D))


def kernel(h, r, pos_t, neg_t, user_entity_embed, relation_embed, W_R):
    r2 = r.astype(jnp.int32).reshape(_B, 1)
    ltri = jnp.tril(jnp.ones((_PB, _PB), jnp.bfloat16))
    dh, dp, dn, brel, bnv = _positions(r2, ltri)

    idx3 = jnp.concatenate([h, pos_t, neg_t]).astype(jnp.int32).reshape(
        _NW, _CHUNKS_PER_W, _IDXW)
    dst3 = jnp.concatenate([dh, dp, dn]).reshape(_NW, _CHUNKS_PER_W, _IDXW)
    rows = _sc_gather_scatter(user_entity_embed, idx3, dst3)
    eh_s = rows[:_P]
    ep_s = rows[_P:2 * _P]
    en_s = rows[2 * _P:]

    out = _grouped_loss(brel.reshape(_NBLK), bnv.reshape(_NBLK),
                        eh_s, ep_s, en_s, W_R, relation_embed)
    o = out[0]
    kg_loss = -(o[0] / _B)
    l2_loss = (o[1] + o[2] + o[3] + o[4]) / (2.0 * _B)
    return kg_loss + 1e-5 * l2_loss


# speedup vs baseline: 2.4897x; 1.4519x over previous
import jax
import jax.numpy as jnp
from jax import lax
from jax.experimental import pallas as pl
from jax.experimental.pallas import tpu as pltpu
from jax.experimental.pallas import tpu_sc as plsc

_B = 16384
_D = 64
_R = 64
_G = 128
_P = 24576
_NBLK = _P // _G

_PB = 1024
_NPB = _B // _PB

_NC = 2
_NS = 16
_NW = _NC * _NS
_IDXW = 128
_N_IDX = 3 * _B
_CHUNKS_PER_W = _N_IDX // (_NW * _IDXW)
_ROWS_PER_W = _CHUNKS_PER_W * _IDXW


def _pos_body(r_ref, dh_ref, dp_ref, dn_ref, brel_ref, bnv_ref,
              cnt_s, pos_s):
    i = pl.program_id(0)

    @pl.when(i == 0)
    def _():
        cnt_s[...] = jnp.zeros_like(cnt_s)

    @pl.when(i < _NPB)
    def _pass_a():
        rb = r_ref[...]
        oh = rb == lax.broadcasted_iota(jnp.int32, (_PB, _R), 1)
        o_f = oh.astype(jnp.float32)
        c = o_f
        s = 1
        while s < _PB:
            c = c + jnp.concatenate(
                [jnp.zeros((s, _R), jnp.float32), c[:_PB - s]], axis=0)
            s *= 2
        carry = cnt_s[...]
        pos = jnp.sum(o_f * (carry + c), axis=1, keepdims=True) - 1.0
        off = pl.multiple_of(i * _PB, _PB)
        pos_s[pl.ds(off, _PB), :] = pos
        cnt_s[...] = carry + c[_PB - 1:_PB, :]

    @pl.when(i >= _NPB)
    def _pass_b():
        j = i - _NPB
        cnt = cnt_s[...]
        padc = jnp.floor((cnt + (_G - 1.0)) / _G) * _G
        tri = (lax.broadcasted_iota(jnp.int32, (_R, _R), 0)
               < lax.broadcasted_iota(jnp.int32, (_R, _R), 1)).astype(jnp.float32)
        starts = jnp.dot(padc, tri, preferred_element_type=jnp.float32)

        rb = r_ref[...]
        oh = rb == lax.broadcasted_iota(jnp.int32, (_PB, _R), 1)
        o_f = oh.astype(jnp.float32)
        segstart = jnp.sum(o_f * starts, axis=1, keepdims=True)
        off = pl.multiple_of(j * _PB, _PB)
        d = segstart + pos_s[pl.ds(off, _PB), :]
        dh_ref[...] = d.astype(jnp.int32)
        dp_ref[...] = (d + float(_P)).astype(jnp.int32)
        dn_ref[...] = (d + float(2 * _P)).astype(jnp.int32)

        @pl.when(j == _NPB - 1)
        def _blocks():
            jr = (lax.broadcasted_iota(jnp.int32, (_NBLK, _R), 0) * _G
                  ).astype(jnp.float32)
            relc = lax.broadcasted_iota(jnp.int32, (_NBLK, _R), 1
                                        ).astype(jnp.float32)
            m = (starts <= jr) & (jr < starts + padc)
            m_f = m.astype(jnp.float32)
            brel_ref[...] = jnp.sum(m_f * relc, axis=1, keepdims=True
                                    ).astype(jnp.int32)
            nv = jnp.clip(cnt - (jr - starts), 0.0, float(_G))
            bnv_ref[...] = jnp.sum(m_f * nv, axis=1, keepdims=True
                                   ).astype(jnp.int32)


def _positions(r2):
    return pl.pallas_call(
        _pos_body,
        grid=(2 * _NPB,),
        in_specs=[
            pl.BlockSpec((_PB, 1), lambda i: (i % _NPB, 0)),
        ],
        out_specs=[
            pl.BlockSpec((_PB, 1), lambda i: (i % _NPB, 0)),
            pl.BlockSpec((_PB, 1), lambda i: (i % _NPB, 0)),
            pl.BlockSpec((_PB, 1), lambda i: (i % _NPB, 0)),
            pl.BlockSpec((_NBLK, 1), lambda i: (0, 0)),
            pl.BlockSpec((_NBLK, 1), lambda i: (0, 0)),
        ],
        out_shape=[
            jax.ShapeDtypeStruct((_B, 1), jnp.int32),
            jax.ShapeDtypeStruct((_B, 1), jnp.int32),
            jax.ShapeDtypeStruct((_B, 1), jnp.int32),
            jax.ShapeDtypeStruct((_NBLK, 1), jnp.int32),
            jax.ShapeDtypeStruct((_NBLK, 1), jnp.int32),
        ],
        scratch_shapes=[
            pltpu.VMEM((1, _R), jnp.float32),
            pltpu.VMEM((_B, 1), jnp.float32),
        ],
    )(r2)


def _sc_gs_body(table_ref, idx_ref, dst_ref, out_ref, idx_v, dst_v, rows_v,
                gsem, ssem):
    wid = lax.axis_index("s") * _NC + lax.axis_index("c")
    pltpu.sync_copy(idx_ref.at[wid], idx_v)
    pltpu.sync_copy(dst_ref.at[wid], dst_v)
    gd = []
    for c in range(_CHUNKS_PER_W):
        gd.append(
            pltpu.async_copy(
                table_ref.at[idx_v.at[c]],
                rows_v.at[pl.ds(c * _IDXW, _IDXW)],
                gsem,
            )
        )
    for d in gd:
        d.wait()
    sd = []
    for c in range(_CHUNKS_PER_W):
        sd.append(
            pltpu.async_copy(
                rows_v.at[pl.ds(c * _IDXW, _IDXW)],
                out_ref.at[dst_v.at[c]],
                ssem,
            )
        )
    for d in sd:
        d.wait()


def _sc_gather_scatter(table, idx3, dst3):
    mesh = plsc.VectorSubcoreMesh(core_axis_name="c", subcore_axis_name="s")
    f = pl.kernel(
        _sc_gs_body,
        out_type=jax.ShapeDtypeStruct((3 * _P, _D), jnp.float32),
        mesh=mesh,
        scratch_types=[
            pltpu.VMEM((_CHUNKS_PER_W, _IDXW), jnp.int32),
            pltpu.VMEM((_CHUNKS_PER_W, _IDXW), jnp.int32),
            pltpu.VMEM((_ROWS_PER_W, _D), jnp.float32),
            pltpu.SemaphoreType.DMA,
            pltpu.SemaphoreType.DMA,
        ],
        compiler_params=pltpu.CompilerParams(use_tc_tiling_on_sc=False),
    )
    return f(table, idx3, dst3)


_KB = 4
_LGRID = _NBLK // _KB
_LROWS = _KB * _G


def _loss_body(brel_ref, bnv_ref, eh_ref, ep_ref, en_ref,
               w0_ref, w1_ref, w2_ref, w3_ref,
               r0_ref, r1_ref, r2_ref, r3_ref, out_ref):
    i = pl.program_id(0)
    w_refs = (w0_ref, w1_ref, w2_ref, w3_ref)
    re_refs = (r0_ref, r1_ref, r2_ref, r3_ref)

    s_ls = jnp.float32(0.0)
    l2h = jnp.float32(0.0)
    l2p = jnp.float32(0.0)
    l2n = jnp.float32(0.0)
    l2re = jnp.float32(0.0)
    for k in range(_KB):
        nv = bnv_ref[i * _KB + k]
        w = w_refs[k][0].astype(jnp.bfloat16)
        sl = pl.ds(k * _G, _G)
        e3 = jnp.concatenate([eh_ref[sl, :], ep_ref[sl, :], en_ref[sl, :]],
                             axis=0)
        rm3 = jnp.dot(e3.astype(jnp.bfloat16), w,
                      preferred_element_type=jnp.float32)
        rm_h = rm3[:_G]
        rm_p = rm3[_G:2 * _G]
        rm_n = rm3[2 * _G:]
        re_row = re_refs[k][0]

        mrow = lax.broadcasted_iota(jnp.int32, (_G, 1), 0) < nv

        a = rm_h + re_row
        pos = jnp.sum(jnp.square(a - rm_p), axis=1, keepdims=True)
        neg = jnp.sum(jnp.square(a - rm_n), axis=1, keepdims=True)
        z = neg - pos
        ls = jnp.minimum(z, 0.0) - jnp.log(1.0 + jnp.exp(-jnp.abs(z)))

        s_ls += jnp.sum(jnp.where(mrow, ls, jnp.zeros_like(ls)))
        m2 = jnp.broadcast_to(mrow, (_G, _D))
        zz = jnp.zeros((_G, _D), jnp.float32)
        l2h += jnp.sum(jnp.where(m2, rm_h * rm_h, zz))
        l2p += jnp.sum(jnp.where(m2, rm_p * rm_p, zz))
        l2n += jnp.sum(jnp.where(m2, rm_n * rm_n, zz))
        l2re += nv.astype(jnp.float32) * jnp.sum(re_row * re_row)

    li = lax.broadcasted_iota(jnp.int32, (8, 128), 1)
    si = lax.broadcasted_iota(jnp.int32, (8, 128), 0)
    row0 = si == 0
    vec = (
        jnp.where(row0 & (li == 0), s_ls, 0.0)
        + jnp.where(row0 & (li == 1), l2h, 0.0)
        + jnp.where(row0 & (li == 2), l2re, 0.0)
        + jnp.where(row0 & (li == 3), l2p, 0.0)
        + jnp.where(row0 & (li == 4), l2n, 0.0)
    )

    @pl.when(i == 0)
    def _():
        out_ref[...] = jnp.zeros_like(out_ref)

    out_ref[...] += vec


def _grouped_loss(brel, bnv, rows, W_R, rel_embed):
    w_specs = [
        pl.BlockSpec((1, _D, _D),
                     lambda i, brel, bnv, k=k: (brel[i * _KB + k], 0, 0))
        for k in range(_KB)
    ]
    re_specs = [
        pl.BlockSpec((1, 1, _D),
                     lambda i, brel, bnv, k=k: (brel[i * _KB + k], 0, 0))
        for k in range(_KB)
    ]
    grid_spec = pltpu.PrefetchScalarGridSpec(
        num_scalar_prefetch=2,
        grid=(_LGRID,),
        in_specs=[
            pl.BlockSpec((_LROWS, _D), lambda i, brel, bnv: (i, 0)),
            pl.BlockSpec((_LROWS, _D), lambda i, brel, bnv: (i + _LGRID, 0)),
            pl.BlockSpec((_LROWS, _D), lambda i, brel, bnv: (i + 2 * _LGRID, 0)),
            *w_specs,
            *re_specs,
        ],
        out_specs=pl.BlockSpec((8, 128), lambda i, brel, bnv: (0, 0)),
    )
    return pl.pallas_call(
        _loss_body,
        grid_spec=grid_spec,
        out_shape=jax.ShapeDtypeStruct((8, 128), jnp.float32),
    )(brel, bnv, rows, rows, rows, W_R, W_R, W_R, W_R,
      *([rel_embed.reshape(_R, 1, _D)] * 4))


def kernel(h, r, pos_t, neg_t, user_entity_embed, relation_embed, W_R):
    r2 = r.astype(jnp.int32).reshape(_B, 1)
    dh, dp, dn, brel, bnv = _positions(r2)

    idx3 = jnp.concatenate([h, pos_t, neg_t]).astype(jnp.int32).reshape(
        _NW, _CHUNKS_PER_W, _IDXW)
    dst3 = jnp.concatenate([dh, dp, dn]).reshape(_NW, _CHUNKS_PER_W, _IDXW)
    rows = _sc_gather_scatter(user_entity_embed, idx3, dst3)

    out = _grouped_loss(brel.reshape(_NBLK), bnv.reshape(_NBLK),
                        rows, W_R, relation_embed)
    o = out[0]
    kg_loss = -(o[0] / _B)
    l2_loss = (o[1] + o[2] + o[3] + o[4]) / (2.0 * _B)
    return kg_loss + 1e-5 * l2_loss


# speedup vs baseline: 2.5421x; 1.0210x over previous
import jax
import jax.numpy as jnp
from jax import lax
from jax.experimental import pallas as pl
from jax.experimental.pallas import tpu as pltpu
from jax.experimental.pallas import tpu_sc as plsc

_B = 16384
_D = 64
_R = 64
_G = 128
_P = 24576
_NBLK = _P // _G

_PB = 1024
_NPB = _B // _PB

_NC = 2
_NS = 16
_NW = _NC * _NS
_IDXW = 128
_N_IDX = 3 * _B
_CHUNKS_PER_W = _N_IDX // (_NW * _IDXW)
_ROWS_PER_W = _CHUNKS_PER_W * _IDXW


def _pos_body(r_ref, dh_ref, dp_ref, dn_ref, brel_ref, bnv_ref,
              cnt_s, pos_s):
    i = pl.program_id(0)

    @pl.when(i == 0)
    def _():
        cnt_s[...] = jnp.zeros_like(cnt_s)

    @pl.when(i < _NPB)
    def _pass_a():
        rb = r_ref[...]
        oh = rb == lax.broadcasted_iota(jnp.int32, (_PB, _R), 1)
        o_f = oh.astype(jnp.float32)
        c = o_f
        s = 1
        while s < _PB:
            c = c + jnp.concatenate(
                [jnp.zeros((s, _R), jnp.float32), c[:_PB - s]], axis=0)
            s *= 2
        carry = cnt_s[...]
        pos = jnp.sum(o_f * (carry + c), axis=1, keepdims=True) - 1.0
        off = pl.multiple_of(i * _PB, _PB)
        pos_s[pl.ds(off, _PB), :] = pos
        cnt_s[...] = carry + c[_PB - 1:_PB, :]

    @pl.when(i >= _NPB)
    def _pass_b():
        j = i - _NPB
        cnt = cnt_s[...]
        padc = jnp.floor((cnt + (_G - 1.0)) / _G) * _G
        tri = (lax.broadcasted_iota(jnp.int32, (_R, _R), 0)
               < lax.broadcasted_iota(jnp.int32, (_R, _R), 1)).astype(jnp.float32)
        starts = jnp.dot(padc, tri, preferred_element_type=jnp.float32)

        rb = r_ref[...]
        oh = rb == lax.broadcasted_iota(jnp.int32, (_PB, _R), 1)
        o_f = oh.astype(jnp.float32)
        segstart = jnp.sum(o_f * starts, axis=1, keepdims=True)
        off = pl.multiple_of(j * _PB, _PB)
        d = segstart + pos_s[pl.ds(off, _PB), :]
        dh_ref[...] = d.astype(jnp.int32)
        dp_ref[...] = (d + float(_P)).astype(jnp.int32)
        dn_ref[...] = (d + float(2 * _P)).astype(jnp.int32)

        @pl.when(j == _NPB - 1)
        def _blocks():
            jr = (lax.broadcasted_iota(jnp.int32, (_NBLK, _R), 0) * _G
                  ).astype(jnp.float32)
            relc = lax.broadcasted_iota(jnp.int32, (_NBLK, _R), 1
                                        ).astype(jnp.float32)
            m = (starts <= jr) & (jr < starts + padc)
            m_f = m.astype(jnp.float32)
            brel_ref[...] = jnp.sum(m_f * relc, axis=1, keepdims=True
                                    ).astype(jnp.int32)
            nv = jnp.clip(cnt - (jr - starts), 0.0, float(_G))
            bnv_ref[...] = jnp.sum(m_f * nv, axis=1, keepdims=True
                                   ).astype(jnp.int32)


def _positions(r2):
    return pl.pallas_call(
        _pos_body,
        grid=(2 * _NPB,),
        in_specs=[
            pl.BlockSpec((_PB, 1), lambda i: (i % _NPB, 0)),
        ],
        out_specs=[
            pl.BlockSpec((_PB, 1), lambda i: (i % _NPB, 0)),
            pl.BlockSpec((_PB, 1), lambda i: (i % _NPB, 0)),
            pl.BlockSpec((_PB, 1), lambda i: (i % _NPB, 0)),
            pl.BlockSpec((_NBLK, 1), lambda i: (0, 0)),
            pl.BlockSpec((_NBLK, 1), lambda i: (0, 0)),
        ],
        out_shape=[
            jax.ShapeDtypeStruct((_B, 1), jnp.int32),
            jax.ShapeDtypeStruct((_B, 1), jnp.int32),
            jax.ShapeDtypeStruct((_B, 1), jnp.int32),
            jax.ShapeDtypeStruct((_NBLK, 1), jnp.int32),
            jax.ShapeDtypeStruct((_NBLK, 1), jnp.int32),
        ],
        scratch_shapes=[
            pltpu.VMEM((1, _R), jnp.float32),
            pltpu.VMEM((_B, 1), jnp.float32),
        ],
    )(r2)


_TPW = _B // _NW


def _sc_gs_body(table_ref, h_ref, p_ref, n_ref, dh_ref, dp_ref, dn_ref,
                out_ref, idx_v, dst_v, rows_v, gsem, ssem):
    wid = lax.axis_index("s") * _NC + lax.axis_index("c")
    base = wid * _TPW
    for t, (iref, dref) in enumerate(
            ((h_ref, dh_ref), (p_ref, dp_ref), (n_ref, dn_ref))):
        pltpu.sync_copy(iref.at[pl.ds(base, _TPW)],
                        idx_v.at[pl.ds(t * _TPW, _TPW)])
        pltpu.sync_copy(dref.at[pl.ds(base, _TPW)],
                        dst_v.at[pl.ds(t * _TPW, _TPW)])
    gd = []
    for c in range(_CHUNKS_PER_W):
        gd.append(
            pltpu.async_copy(
                table_ref.at[idx_v.at[pl.ds(c * _IDXW, _IDXW)]],
                rows_v.at[pl.ds(c * _IDXW, _IDXW)],
                gsem,
            )
        )
    for d in gd:
        d.wait()
    sd = []
    for c in range(_CHUNKS_PER_W):
        sd.append(
            pltpu.async_copy(
                rows_v.at[pl.ds(c * _IDXW, _IDXW)],
                out_ref.at[dst_v.at[pl.ds(c * _IDXW, _IDXW)]],
                ssem,
            )
        )
    for d in sd:
        d.wait()


def _sc_gather_scatter(table, h, p, n, dh, dp, dn):
    mesh = plsc.VectorSubcoreMesh(core_axis_name="c", subcore_axis_name="s")
    f = pl.kernel(
        _sc_gs_body,
        out_type=jax.ShapeDtypeStruct((3 * _P, _D), jnp.float32),
        mesh=mesh,
        scratch_types=[
            pltpu.VMEM((_ROWS_PER_W,), jnp.int32),
            pltpu.VMEM((_ROWS_PER_W,), jnp.int32),
            pltpu.VMEM((_ROWS_PER_W, _D), jnp.float32),
            pltpu.SemaphoreType.DMA,
            pltpu.SemaphoreType.DMA,
        ],
        compiler_params=pltpu.CompilerParams(use_tc_tiling_on_sc=False),
    )
    return f(table, h, p, n, dh, dp, dn)


_KB = 4
_LGRID = _NBLK // _KB
_LROWS = _KB * _G


def _loss_body(brel_ref, bnv_ref, eh_ref, ep_ref, en_ref,
               w0_ref, w1_ref, w2_ref, w3_ref,
               r0_ref, r1_ref, r2_ref, r3_ref, out_ref, loss_ref):
    i = pl.program_id(0)
    w_refs = (w0_ref, w1_ref, w2_ref, w3_ref)
    re_refs = (r0_ref, r1_ref, r2_ref, r3_ref)

    s_ls = jnp.float32(0.0)
    l2h = jnp.float32(0.0)
    l2p = jnp.float32(0.0)
    l2n = jnp.float32(0.0)
    l2re = jnp.float32(0.0)
    for k in range(_KB):
        nv = bnv_ref[i * _KB + k]
        w = w_refs[k][0].astype(jnp.bfloat16)
        sl = pl.ds(k * _G, _G)
        e3 = jnp.concatenate([eh_ref[sl, :], ep_ref[sl, :], en_ref[sl, :]],
                             axis=0)
        rm3 = jnp.dot(e3.astype(jnp.bfloat16), w,
                      preferred_element_type=jnp.float32)
        rm_h = rm3[:_G]
        rm_p = rm3[_G:2 * _G]
        rm_n = rm3[2 * _G:]
        re_row = re_refs[k][0]

        mrow = lax.broadcasted_iota(jnp.int32, (_G, 1), 0) < nv

        a = rm_h + re_row
        pos = jnp.sum(jnp.square(a - rm_p), axis=1, keepdims=True)
        neg = jnp.sum(jnp.square(a - rm_n), axis=1, keepdims=True)
        z = neg - pos
        ls = jnp.minimum(z, 0.0) - jnp.log(1.0 + jnp.exp(-jnp.abs(z)))

        s_ls += jnp.sum(jnp.where(mrow, ls, jnp.zeros_like(ls)))
        m2 = jnp.broadcast_to(mrow, (_G, _D))
        zz = jnp.zeros((_G, _D), jnp.float32)
        l2h += jnp.sum(jnp.where(m2, rm_h * rm_h, zz))
        l2p += jnp.sum(jnp.where(m2, rm_p * rm_p, zz))
        l2n += jnp.sum(jnp.where(m2, rm_n * rm_n, zz))
        l2re += nv.astype(jnp.float32) * jnp.sum(re_row * re_row)

    li = lax.broadcasted_iota(jnp.int32, (8, 128), 1)
    si = lax.broadcasted_iota(jnp.int32, (8, 128), 0)
    row0 = si == 0
    vec = (
        jnp.where(row0 & (li == 0), s_ls, 0.0)
        + jnp.where(row0 & (li == 1), l2h, 0.0)
        + jnp.where(row0 & (li == 2), l2re, 0.0)
        + jnp.where(row0 & (li == 3), l2p, 0.0)
        + jnp.where(row0 & (li == 4), l2n, 0.0)
    )

    @pl.when(i == 0)
    def _():
        out_ref[...] = jnp.zeros_like(out_ref)

    out_ref[...] += vec

    @pl.when(i == _LGRID - 1)
    def _finish():
        acc = out_ref[...]
        t_ls = jnp.sum(jnp.where(row0 & (li == 0), acc, 0.0))
        t_l2 = jnp.sum(jnp.where(row0 & (li >= 1) & (li <= 4), acc, 0.0))
        loss_ref[...] = (-(t_ls / _B)
                         + 1e-5 * (t_l2 / (2.0 * _B))) * jnp.ones((1, 1),
                                                                  jnp.float32)


def _grouped_loss(brel, bnv, rows, W_R, rel_embed):
    w_specs = [
        pl.BlockSpec((1, _D, _D),
                     lambda i, brel, bnv, k=k: (brel[i * _KB + k], 0, 0))
        for k in range(_KB)
    ]
    re_specs = [
        pl.BlockSpec((1, 1, _D),
                     lambda i, brel, bnv, k=k: (brel[i * _KB + k], 0, 0))
        for k in range(_KB)
    ]
    grid_spec = pltpu.PrefetchScalarGridSpec(
        num_scalar_prefetch=2,
        grid=(_LGRID,),
        in_specs=[
            pl.BlockSpec((_LROWS, _D), lambda i, brel, bnv: (i, 0)),
            pl.BlockSpec((_LROWS, _D), lambda i, brel, bnv: (i + _LGRID, 0)),
            pl.BlockSpec((_LROWS, _D), lambda i, brel, bnv: (i + 2 * _LGRID, 0)),
            *w_specs,
            *re_specs,
        ],
        out_specs=[
            pl.BlockSpec((8, 128), lambda i, brel, bnv: (0, 0)),
            pl.BlockSpec((1, 1), lambda i, brel, bnv: (0, 0)),
        ],
    )
    return pl.pallas_call(
        _loss_body,
        grid_spec=grid_spec,
        out_shape=[
            jax.ShapeDtypeStruct((8, 128), jnp.float32),
            jax.ShapeDtypeStruct((1, 1), jnp.float32),
        ],
    )(brel, bnv, rows, rows, rows, W_R, W_R, W_R, W_R,
      *([rel_embed.reshape(_R, 1, _D)] * 4))


def kernel(h, r, pos_t, neg_t, user_entity_embed, relation_embed, W_R):
    r2 = r.astype(jnp.int32).reshape(_B, 1)
    dh, dp, dn, brel, bnv = _positions(r2)

    rows = _sc_gather_scatter(
        user_entity_embed,
        h.astype(jnp.int32), pos_t.astype(jnp.int32), neg_t.astype(jnp.int32),
        dh.reshape(_B), dp.reshape(_B), dn.reshape(_B))

    _, loss = _grouped_loss(brel.reshape(_NBLK), bnv.reshape(_NBLK),
                            rows, W_R, relation_embed)
    return loss[0, 0]


# speedup vs baseline: 2.7287x; 1.0734x over previous
import jax
import jax.numpy as jnp
from jax import lax
from jax.experimental import pallas as pl
from jax.experimental.pallas import tpu as pltpu
from jax.experimental.pallas import tpu_sc as plsc

_B = 16384
_D = 64
_R = 64
_G = 128
_P = 24576
_NBLK = _P // _G

_PB = 1024
_NPB = _B // _PB

_NC = 2
_NS = 16
_NW = _NC * _NS
_IDXW = 128
_N_IDX = 3 * _B
_CHUNKS_PER_W = _N_IDX // (_NW * _IDXW)
_ROWS_PER_W = _CHUNKS_PER_W * _IDXW


def _pos_body(r_ref, dh_ref, dp_ref, dn_ref, brel_ref, bnv_ref,
              cnt_s, pos_s):
    i = pl.program_id(0)

    @pl.when(i == 0)
    def _():
        cnt_s[...] = jnp.zeros_like(cnt_s)

    @pl.when(i < _NPB)
    def _pass_a():
        rb = r_ref[...]
        oh = rb == lax.broadcasted_iota(jnp.int32, (_PB, _R), 1)
        o_f = oh.astype(jnp.float32)
        c = o_f
        s = 1
        while s < _PB:
            c = c + jnp.concatenate(
                [jnp.zeros((s, _R), jnp.float32), c[:_PB - s]], axis=0)
            s *= 2
        carry = cnt_s[...]
        pos = jnp.sum(o_f * (carry + c), axis=1, keepdims=True) - 1.0
        off = pl.multiple_of(i * _PB, _PB)
        pos_s[pl.ds(off, _PB), :] = pos
        cnt_s[...] = carry + c[_PB - 1:_PB, :]

    @pl.when(i >= _NPB)
    def _pass_b():
        j = i - _NPB
        cnt = cnt_s[...]
        padc = jnp.floor((cnt + (_G - 1.0)) / _G) * _G
        tri = (lax.broadcasted_iota(jnp.int32, (_R, _R), 0)
               < lax.broadcasted_iota(jnp.int32, (_R, _R), 1)).astype(jnp.float32)
        starts = jnp.dot(padc, tri, preferred_element_type=jnp.float32)

        rb = r_ref[...]
        oh = rb == lax.broadcasted_iota(jnp.int32, (_PB, _R), 1)
        o_f = oh.astype(jnp.float32)
        segstart = jnp.sum(o_f * starts, axis=1, keepdims=True)
        off = pl.multiple_of(j * _PB, _PB)
        d = segstart + pos_s[pl.ds(off, _PB), :]
        dh_ref[...] = d.astype(jnp.int32)
        dp_ref[...] = (d + float(_P)).astype(jnp.int32)
        dn_ref[...] = (d + float(2 * _P)).astype(jnp.int32)

        @pl.when(j == _NPB - 1)
        def _blocks():
            jr = (lax.broadcasted_iota(jnp.int32, (_NBLK, _R), 0) * _G
                  ).astype(jnp.float32)
            relc = lax.broadcasted_iota(jnp.int32, (_NBLK, _R), 1
                                        ).astype(jnp.float32)
            m = (starts <= jr) & (jr < starts + padc)
            m_f = m.astype(jnp.float32)
            brel_ref[...] = jnp.sum(m_f * relc, axis=1, keepdims=True
                                    ).astype(jnp.int32)
            nv = jnp.clip(cnt - (jr - starts), 0.0, float(_G))
            bnv_ref[...] = jnp.sum(m_f * nv, axis=1, keepdims=True
                                   ).astype(jnp.int32)


def _positions(r2):
    return pl.pallas_call(
        _pos_body,
        grid=(2 * _NPB,),
        in_specs=[
            pl.BlockSpec((_PB, 1), lambda i: (i % _NPB, 0)),
        ],
        out_specs=[
            pl.BlockSpec((_PB, 1), lambda i: (i % _NPB, 0)),
            pl.BlockSpec((_PB, 1), lambda i: (i % _NPB, 0)),
            pl.BlockSpec((_PB, 1), lambda i: (i % _NPB, 0)),
            pl.BlockSpec((_NBLK, 1), lambda i: (0, 0)),
            pl.BlockSpec((_NBLK, 1), lambda i: (0, 0)),
        ],
        out_shape=[
            jax.ShapeDtypeStruct((_B, 1), jnp.int32),
            jax.ShapeDtypeStruct((_B, 1), jnp.int32),
            jax.ShapeDtypeStruct((_B, 1), jnp.int32),
            jax.ShapeDtypeStruct((_NBLK, 1), jnp.int32),
            jax.ShapeDtypeStruct((_NBLK, 1), jnp.int32),
        ],
        scratch_shapes=[
            pltpu.VMEM((1, _R), jnp.float32),
            pltpu.VMEM((_B, 1), jnp.float32),
        ],
    )(r2)


_TPW = _B // _NW


def _sc_gs_body(table_ref, h_ref, p_ref, n_ref, dh_ref, dp_ref, dn_ref,
                out_ref, idx_v, dst_v, rows_v, gsem, ssem):
    wid = lax.axis_index("s") * _NC + lax.axis_index("c")
    base = wid * _TPW
    for t, (iref, dref) in enumerate(
            ((h_ref, dh_ref), (p_ref, dp_ref), (n_ref, dn_ref))):
        pltpu.sync_copy(iref.at[pl.ds(base, _TPW)],
                        idx_v.at[pl.ds(t * _TPW, _TPW)])
        pltpu.sync_copy(dref.at[pl.ds(base, _TPW)],
                        dst_v.at[pl.ds(t * _TPW, _TPW)])
    gd = []
    for c in range(_CHUNKS_PER_W):
        gd.append(
            pltpu.async_copy(
                table_ref.at[idx_v.at[pl.ds(c * _IDXW, _IDXW)]],
                rows_v.at[pl.ds(c * _IDXW, _IDXW)],
                gsem,
            )
        )
    for d in gd:
        d.wait()
    sd = []
    for c in range(_CHUNKS_PER_W):
        sd.append(
            pltpu.async_copy(
                rows_v.at[pl.ds(c * _IDXW, _IDXW)],
                out_ref.at[dst_v.at[pl.ds(c * _IDXW, _IDXW)]],
                ssem,
            )
        )
    for d in sd:
        d.wait()


def _sc_gather_scatter(table, h, p, n, dh, dp, dn):
    mesh = plsc.VectorSubcoreMesh(core_axis_name="c", subcore_axis_name="s")
    f = pl.kernel(
        _sc_gs_body,
        out_type=jax.ShapeDtypeStruct((3 * _P, _D), jnp.float32),
        mesh=mesh,
        scratch_types=[
            pltpu.VMEM((_ROWS_PER_W,), jnp.int32),
            pltpu.VMEM((_ROWS_PER_W,), jnp.int32),
            pltpu.VMEM((_ROWS_PER_W, _D), jnp.float32),
            pltpu.SemaphoreType.DMA,
            pltpu.SemaphoreType.DMA,
        ],
        compiler_params=pltpu.CompilerParams(use_tc_tiling_on_sc=False),
    )
    return f(table, h, p, n, dh, dp, dn)


_KB = 4
_LGRID = _NBLK // _KB
_LROWS = _KB * _G


def _loss_body(brel_ref, bnv_ref, eh_ref, ep_ref, en_ref,
               w0_ref, w1_ref, w2_ref, w3_ref,
               r0_ref, r1_ref, r2_ref, r3_ref, out_ref, loss_ref):
    i = pl.program_id(0)
    w_refs = (w0_ref, w1_ref, w2_ref, w3_ref)
    re_refs = (r0_ref, r1_ref, r2_ref, r3_ref)
    _H = _G // 2

    s_ls = jnp.float32(0.0)
    l2h = jnp.float32(0.0)
    l2p = jnp.float32(0.0)
    l2n = jnp.float32(0.0)
    l2re = jnp.float32(0.0)
    for k in range(_KB):
        nv = bnv_ref[i * _KB + k]
        w = w_refs[k][0].astype(jnp.bfloat16)
        zzb = jnp.zeros((_D, _D), jnp.bfloat16)
        w2 = jnp.concatenate(
            [jnp.concatenate([w, zzb], axis=1),
             jnp.concatenate([zzb, w], axis=1)], axis=0)
        sl = pl.ds(k * _H, _H)
        e3 = jnp.concatenate([eh_ref[sl, :], ep_ref[sl, :], en_ref[sl, :]],
                             axis=0)
        rm3 = jnp.dot(e3.astype(jnp.bfloat16), w2,
                      preferred_element_type=jnp.float32)
        re_row = re_refs[k][0]
        re2 = jnp.concatenate([re_row, re_row], axis=1)

        a = rm3[:_H] + re2
        dp_ = a - rm3[_H:2 * _H]
        dn_ = a - rm3[2 * _H:]
        dp2 = dp_ * dp_
        dn2 = dn_ * dn_
        pos_e = jnp.sum(dp2[:, :_D], axis=1, keepdims=True)
        pos_o = jnp.sum(dp2[:, _D:], axis=1, keepdims=True)
        neg_e = jnp.sum(dn2[:, :_D], axis=1, keepdims=True)
        neg_o = jnp.sum(dn2[:, _D:], axis=1, keepdims=True)
        pos = jnp.concatenate([pos_e, pos_o], axis=1)
        neg = jnp.concatenate([neg_e, neg_o], axis=1)
        z = neg - pos
        ls = jnp.minimum(z, 0.0) - jnp.log(1.0 + jnp.exp(-jnp.abs(z)))

        j2 = lax.broadcasted_iota(jnp.int32, (_H, 2), 0) * 2 \
            + lax.broadcasted_iota(jnp.int32, (_H, 2), 1)
        mrow = j2 < nv
        s_ls += jnp.sum(jnp.where(mrow, ls, jnp.zeros_like(ls)))

        je = lax.broadcasted_iota(jnp.int32, (_H, 2 * _D), 0) * 2 \
            + (lax.broadcasted_iota(jnp.int32, (_H, 2 * _D), 1) // _D)
        m2 = je < nv
        zz = jnp.zeros((_H, 2 * _D), jnp.float32)
        rm_h = rm3[:_H]
        rm_p = rm3[_H:2 * _H]
        rm_n = rm3[2 * _H:]
        l2h += jnp.sum(jnp.where(m2, rm_h * rm_h, zz))
        l2p += jnp.sum(jnp.where(m2, rm_p * rm_p, zz))
        l2n += jnp.sum(jnp.where(m2, rm_n * rm_n, zz))
        l2re += nv.astype(jnp.float32) * jnp.sum(re_row * re_row)

    li = lax.broadcasted_iota(jnp.int32, (8, 128), 1)
    si = lax.broadcasted_iota(jnp.int32, (8, 128), 0)
    row0 = si == 0
    vec = (
        jnp.where(row0 & (li == 0), s_ls, 0.0)
        + jnp.where(row0 & (li == 1), l2h, 0.0)
        + jnp.where(row0 & (li == 2), l2re, 0.0)
        + jnp.where(row0 & (li == 3), l2p, 0.0)
        + jnp.where(row0 & (li == 4), l2n, 0.0)
    )

    @pl.when(i == 0)
    def _():
        out_ref[...] = jnp.zeros_like(out_ref)

    out_ref[...] += vec

    @pl.when(i == _LGRID - 1)
    def _finish():
        acc = out_ref[...]
        t_ls = jnp.sum(jnp.where(row0 & (li == 0), acc, 0.0))
        t_l2 = jnp.sum(jnp.where(row0 & (li >= 1) & (li <= 4), acc, 0.0))
        loss_ref[...] = (-(t_ls / _B)
                         + 1e-5 * (t_l2 / (2.0 * _B))) * jnp.ones((1, 1),
                                                                  jnp.float32)


def _grouped_loss(brel, bnv, rows, W_R, rel_embed):
    w_specs = [
        pl.BlockSpec((1, _D, _D),
                     lambda i, brel, bnv, k=k: (brel[i * _KB + k], 0, 0))
        for k in range(_KB)
    ]
    re_specs = [
        pl.BlockSpec((1, 1, _D),
                     lambda i, brel, bnv, k=k: (brel[i * _KB + k], 0, 0))
        for k in range(_KB)
    ]
    grid_spec = pltpu.PrefetchScalarGridSpec(
        num_scalar_prefetch=2,
        grid=(_LGRID,),
        in_specs=[
            pl.BlockSpec((_LROWS // 2, 2 * _D), lambda i, brel, bnv: (i, 0)),
            pl.BlockSpec((_LROWS // 2, 2 * _D),
                         lambda i, brel, bnv: (i + _LGRID, 0)),
            pl.BlockSpec((_LROWS // 2, 2 * _D),
                         lambda i, brel, bnv: (i + 2 * _LGRID, 0)),
            *w_specs,
            *re_specs,
        ],
        out_specs=[
            pl.BlockSpec((8, 128), lambda i, brel, bnv: (0, 0)),
            pl.BlockSpec((1, 1), lambda i, brel, bnv: (0, 0)),
        ],
    )
    return pl.pallas_call(
        _loss_body,
        grid_spec=grid_spec,
        out_shape=[
            jax.ShapeDtypeStruct((8, 128), jnp.float32),
            jax.ShapeDtypeStruct((1, 1), jnp.float32),
        ],
    )(brel, bnv, rows, rows, rows, W_R, W_R, W_R, W_R,
      *([rel_embed.reshape(_R, 1, _D)] * 4))


def kernel(h, r, pos_t, neg_t, user_entity_embed, relation_embed, W_R):
    r2 = r.astype(jnp.int32).reshape(_B, 1)
    dh, dp, dn, brel, bnv = _positions(r2)

    rows = _sc_gather_scatter(
        user_entity_embed,
        h.astype(jnp.int32), pos_t.astype(jnp.int32), neg_t.astype(jnp.int32),
        dh.reshape(_B), dp.reshape(_B), dn.reshape(_B))

    rows_packed = rows.reshape(3 * _P * _D // (2 * _D), 2 * _D)
    _, loss = _grouped_loss(brel.reshape(_NBLK), bnv.reshape(_NBLK),
                            rows_packed, W_R, relation_embed)
    return loss[0, 0]


# speedup vs baseline: 2.7546x; 1.0095x over previous
import jax
import jax.numpy as jnp
from jax import lax
from jax.experimental import pallas as pl
from jax.experimental.pallas import tpu as pltpu
from jax.experimental.pallas import tpu_sc as plsc

_B = 16384
_D = 64
_R = 64
_G = 128
_P = 24576
_NBLK = _P // _G

_PB = 2048
_NPB = _B // _PB

_NC = 2
_NS = 16
_NW = _NC * _NS
_IDXW = 128
_N_IDX = 3 * _B
_CHUNKS_PER_W = _N_IDX // (_NW * _IDXW)
_ROWS_PER_W = _CHUNKS_PER_W * _IDXW


def _pos_body(r_ref, dh_ref, dp_ref, dn_ref, brel_ref, bnv_ref,
              cnt_s, pos_s):
    i = pl.program_id(0)

    @pl.when(i == 0)
    def _():
        cnt_s[...] = jnp.zeros_like(cnt_s)

    @pl.when(i < _NPB)
    def _pass_a():
        rb = r_ref[...]
        oh = rb == lax.broadcasted_iota(jnp.int32, (_PB, _R), 1)
        o_f = oh.astype(jnp.float32)
        c = o_f
        s = 1
        while s < _PB:
            c = c + jnp.concatenate(
                [jnp.zeros((s, _R), jnp.float32), c[:_PB - s]], axis=0)
            s *= 2
        carry = cnt_s[...]
        pos = jnp.sum(o_f * (carry + c), axis=1, keepdims=True) - 1.0
        off = pl.multiple_of(i * _PB, _PB)
        pos_s[pl.ds(off, _PB), :] = pos
        cnt_s[...] = carry + c[_PB - 1:_PB, :]

    @pl.when(i >= _NPB)
    def _pass_b():
        j = i - _NPB
        cnt = cnt_s[...]
        padc = jnp.floor((cnt + (_G - 1.0)) / _G) * _G
        tri = (lax.broadcasted_iota(jnp.int32, (_R, _R), 0)
               < lax.broadcasted_iota(jnp.int32, (_R, _R), 1)).astype(jnp.float32)
        starts = jnp.dot(padc, tri, preferred_element_type=jnp.float32)

        rb = r_ref[...]
        oh = rb == lax.broadcasted_iota(jnp.int32, (_PB, _R), 1)
        o_f = oh.astype(jnp.float32)
        segstart = jnp.sum(o_f * starts, axis=1, keepdims=True)
        off = pl.multiple_of(j * _PB, _PB)
        d = segstart + pos_s[pl.ds(off, _PB), :]
        dh_ref[...] = d.astype(jnp.int32)
        dp_ref[...] = (d + float(_P)).astype(jnp.int32)
        dn_ref[...] = (d + float(2 * _P)).astype(jnp.int32)

        @pl.when(j == _NPB - 1)
        def _blocks():
            jr = (lax.broadcasted_iota(jnp.int32, (_NBLK, _R), 0) * _G
                  ).astype(jnp.float32)
            relc = lax.broadcasted_iota(jnp.int32, (_NBLK, _R), 1
                                        ).astype(jnp.float32)
            m = (starts <= jr) & (jr < starts + padc)
            m_f = m.astype(jnp.float32)
            brel_ref[...] = jnp.sum(m_f * relc, axis=1, keepdims=True
                                    ).astype(jnp.int32)
            nv = jnp.clip(cnt - (jr - starts), 0.0, float(_G))
            bnv_ref[...] = jnp.sum(m_f * nv, axis=1, keepdims=True
                                   ).astype(jnp.int32)


def _positions(r2):
    return pl.pallas_call(
        _pos_body,
        grid=(2 * _NPB,),
        in_specs=[
            pl.BlockSpec((_PB, 1), lambda i: (i % _NPB, 0)),
        ],
        out_specs=[
            pl.BlockSpec((_PB, 1), lambda i: (i % _NPB, 0)),
            pl.BlockSpec((_PB, 1), lambda i: (i % _NPB, 0)),
            pl.BlockSpec((_PB, 1), lambda i: (i % _NPB, 0)),
            pl.BlockSpec((_NBLK, 1), lambda i: (0, 0)),
            pl.BlockSpec((_NBLK, 1), lambda i: (0, 0)),
        ],
        out_shape=[
            jax.ShapeDtypeStruct((_B, 1), jnp.int32),
            jax.ShapeDtypeStruct((_B, 1), jnp.int32),
            jax.ShapeDtypeStruct((_B, 1), jnp.int32),
            jax.ShapeDtypeStruct((_NBLK, 1), jnp.int32),
            jax.ShapeDtypeStruct((_NBLK, 1), jnp.int32),
        ],
        scratch_shapes=[
            pltpu.VMEM((1, _R), jnp.float32),
            pltpu.VMEM((_B, 1), jnp.float32),
        ],
    )(r2)


_TPW = _B // _NW


def _sc_gs_body(table_ref, h_ref, p_ref, n_ref, dh_ref, dp_ref, dn_ref,
                out_ref, idx_v, dst_v, rows_v, gsem, ssem):
    wid = lax.axis_index("s") * _NC + lax.axis_index("c")
    base = wid * _TPW
    for t, (iref, dref) in enumerate(
            ((h_ref, dh_ref), (p_ref, dp_ref), (n_ref, dn_ref))):
        pltpu.sync_copy(iref.at[pl.ds(base, _TPW)],
                        idx_v.at[pl.ds(t * _TPW, _TPW)])
        pltpu.sync_copy(dref.at[pl.ds(base, _TPW)],
                        dst_v.at[pl.ds(t * _TPW, _TPW)])
    gd = []
    for c in range(_CHUNKS_PER_W):
        gd.append(
            pltpu.async_copy(
                table_ref.at[idx_v.at[pl.ds(c * _IDXW, _IDXW)]],
                rows_v.at[pl.ds(c * _IDXW, _IDXW)],
                gsem,
            )
        )
    for d in gd:
        d.wait()
    sd = []
    for c in range(_CHUNKS_PER_W):
        sd.append(
            pltpu.async_copy(
                rows_v.at[pl.ds(c * _IDXW, _IDXW)],
                out_ref.at[dst_v.at[pl.ds(c * _IDXW, _IDXW)]],
                ssem,
            )
        )
    for d in sd:
        d.wait()


def _sc_gather_scatter(table, h, p, n, dh, dp, dn):
    mesh = plsc.VectorSubcoreMesh(core_axis_name="c", subcore_axis_name="s")
    f = pl.kernel(
        _sc_gs_body,
        out_type=jax.ShapeDtypeStruct((3 * _P, _D), jnp.float32),
        mesh=mesh,
        scratch_types=[
            pltpu.VMEM((_ROWS_PER_W,), jnp.int32),
            pltpu.VMEM((_ROWS_PER_W,), jnp.int32),
            pltpu.VMEM((_ROWS_PER_W, _D), jnp.float32),
            pltpu.SemaphoreType.DMA,
            pltpu.SemaphoreType.DMA,
        ],
        compiler_params=pltpu.CompilerParams(use_tc_tiling_on_sc=False),
    )
    return f(table, h, p, n, dh, dp, dn)


_KB = 4
_LGRID = _NBLK // _KB
_LROWS = _KB * _G


def _loss_body(brel_ref, bnv_ref, eh_ref, ep_ref, en_ref,
               w0_ref, w1_ref, w2_ref, w3_ref,
               r0_ref, r1_ref, r2_ref, r3_ref, out_ref, loss_ref):
    i = pl.program_id(0)
    w_refs = (w0_ref, w1_ref, w2_ref, w3_ref)
    re_refs = (r0_ref, r1_ref, r2_ref, r3_ref)
    _H = _G // 2

    s_ls = jnp.float32(0.0)
    l2h = jnp.float32(0.0)
    l2p = jnp.float32(0.0)
    l2n = jnp.float32(0.0)
    l2re = jnp.float32(0.0)
    for k in range(_KB):
        nv = bnv_ref[i * _KB + k]
        w = w_refs[k][0].astype(jnp.bfloat16)
        zzb = jnp.zeros((_D, _D), jnp.bfloat16)
        w2 = jnp.concatenate(
            [jnp.concatenate([w, zzb], axis=1),
             jnp.concatenate([zzb, w], axis=1)], axis=0)
        sl = pl.ds(k * _H, _H)
        e3 = jnp.concatenate([eh_ref[sl, :], ep_ref[sl, :], en_ref[sl, :]],
                             axis=0)
        rm3 = jnp.dot(e3.astype(jnp.bfloat16), w2,
                      preferred_element_type=jnp.float32)
        re_row = re_refs[k][0]
        re2 = jnp.concatenate([re_row, re_row], axis=1)

        a = rm3[:_H] + re2
        dp_ = a - rm3[_H:2 * _H]
        dn_ = a - rm3[2 * _H:]
        dp2 = dp_ * dp_
        dn2 = dn_ * dn_
        pos_e = jnp.sum(dp2[:, :_D], axis=1, keepdims=True)
        pos_o = jnp.sum(dp2[:, _D:], axis=1, keepdims=True)
        neg_e = jnp.sum(dn2[:, :_D], axis=1, keepdims=True)
        neg_o = jnp.sum(dn2[:, _D:], axis=1, keepdims=True)
        pos = jnp.concatenate([pos_e, pos_o], axis=1)
        neg = jnp.concatenate([neg_e, neg_o], axis=1)
        z = neg - pos
        ls = jnp.minimum(z, 0.0) - jnp.log(1.0 + jnp.exp(-jnp.abs(z)))

        j2 = lax.broadcasted_iota(jnp.int32, (_H, 2), 0) * 2 \
            + lax.broadcasted_iota(jnp.int32, (_H, 2), 1)
        mrow = j2 < nv
        s_ls += jnp.sum(jnp.where(mrow, ls, jnp.zeros_like(ls)))

        je = lax.broadcasted_iota(jnp.int32, (_H, 2 * _D), 0) * 2 \
            + (lax.broadcasted_iota(jnp.int32, (_H, 2 * _D), 1) // _D)
        m2 = je < nv
        zz = jnp.zeros((_H, 2 * _D), jnp.float32)
        rm_h = rm3[:_H]
        rm_p = rm3[_H:2 * _H]
        rm_n = rm3[2 * _H:]
        l2h += jnp.sum(jnp.where(m2, rm_h * rm_h, zz))
        l2p += jnp.sum(jnp.where(m2, rm_p * rm_p, zz))
        l2n += jnp.sum(jnp.where(m2, rm_n * rm_n, zz))
        l2re += nv.astype(jnp.float32) * jnp.sum(re_row * re_row)

    li = lax.broadcasted_iota(jnp.int32, (8, 128), 1)
    si = lax.broadcasted_iota(jnp.int32, (8, 128), 0)
    row0 = si == 0
    vec = (
        jnp.where(row0 & (li == 0), s_ls, 0.0)
        + jnp.where(row0 & (li == 1), l2h, 0.0)
        + jnp.where(row0 & (li == 2), l2re, 0.0)
        + jnp.where(row0 & (li == 3), l2p, 0.0)
        + jnp.where(row0 & (li == 4), l2n, 0.0)
    )

    @pl.when(i == 0)
    def _():
        out_ref[...] = jnp.zeros_like(out_ref)

    out_ref[...] += vec

    @pl.when(i == _LGRID - 1)
    def _finish():
        acc = out_ref[...]
        t_ls = jnp.sum(jnp.where(row0 & (li == 0), acc, 0.0))
        t_l2 = jnp.sum(jnp.where(row0 & (li >= 1) & (li <= 4), acc, 0.0))
        loss_ref[...] = (-(t_ls / _B)
                         + 1e-5 * (t_l2 / (2.0 * _B))) * jnp.ones((1, 1),
                                                                  jnp.float32)


def _grouped_loss(brel, bnv, rows, W_R, rel_embed):
    w_specs = [
        pl.BlockSpec((1, _D, _D),
                     lambda i, brel, bnv, k=k: (brel[i * _KB + k], 0, 0))
        for k in range(_KB)
    ]
    re_specs = [
        pl.BlockSpec((1, 1, _D),
                     lambda i, brel, bnv, k=k: (brel[i * _KB + k], 0, 0))
        for k in range(_KB)
    ]
    grid_spec = pltpu.PrefetchScalarGridSpec(
        num_scalar_prefetch=2,
        grid=(_LGRID,),
        in_specs=[
            pl.BlockSpec((_LROWS // 2, 2 * _D), lambda i, brel, bnv: (i, 0)),
            pl.BlockSpec((_LROWS // 2, 2 * _D),
                         lambda i, brel, bnv: (i + _LGRID, 0)),
            pl.BlockSpec((_LROWS // 2, 2 * _D),
                         lambda i, brel, bnv: (i + 2 * _LGRID, 0)),
            *w_specs,
            *re_specs,
        ],
        out_specs=[
            pl.BlockSpec((8, 128), lambda i, brel, bnv: (0, 0)),
            pl.BlockSpec((1, 1), lambda i, brel, bnv: (0, 0)),
        ],
    )
    return pl.pallas_call(
        _loss_body,
        grid_spec=grid_spec,
        out_shape=[
            jax.ShapeDtypeStruct((8, 128), jnp.float32),
            jax.ShapeDtypeStruct((1, 1), jnp.float32),
        ],
    )(brel, bnv, rows, rows, rows, W_R, W_R, W_R, W_R,
      *([rel_embed.reshape(_R, 1, _D)] * 4))


def kernel(h, r, pos_t, neg_t, user_entity_embed, relation_embed, W_R):
    r2 = r.astype(jnp.int32).reshape(_B, 1)
    dh, dp, dn, brel, bnv = _positions(r2)

    rows = _sc_gather_scatter(
        user_entity_embed,
        h.astype(jnp.int32), pos_t.astype(jnp.int32), neg_t.astype(jnp.int32),
        dh.reshape(_B), dp.reshape(_B), dn.reshape(_B))

    rows_packed = rows.reshape(3 * _P * _D // (2 * _D), 2 * _D)
    _, loss = _grouped_loss(brel.reshape(_NBLK), bnv.reshape(_NBLK),
                            rows_packed, W_R, relation_embed)
    return loss[0, 0]
